# Initial kernel scaffold; baseline (speedup 1.0000x reference)
import jax
import jax.numpy as jnp
from jax.experimental import pallas as pl


def kernel(edge_index, user_table, item_table):
    raise NotImplementedError("write your pallas kernel here")


# speedup vs baseline: 58.7459x; 58.7459x over previous
import functools

import jax
import jax.numpy as jnp
from jax import lax
from jax.experimental import pallas as pl
from jax.experimental.pallas import tpu as pltpu
from jax.experimental.pallas import tpu_sc as plsc

N_NODES = 50000
E = 800000
D = 64
NC = 2
NS = 16
NW = NC * NS
L = 16

NB = 51200
NPT = NB // NW
EPT_H = 25008
EPAD = EPT_H * NW
EPT = E // NS
CH = 2000

_mesh = plsc.VectorSubcoreMesh(
    core_axis_name="c", subcore_axis_name="s", num_cores=NC, num_subcores=NS)


def _wid():
    return lax.axis_index("s") * NC + lax.axis_index("c")


@functools.partial(
    pl.kernel,
    out_type=jax.ShapeDtypeStruct((NW, NB), jnp.float32),
    mesh=_mesh,
    compiler_params=pltpu.CompilerParams(needs_layout_passes=False),
    scratch_types=[
        pltpu.VMEM((NB,), jnp.float32),
        pltpu.VMEM((EPT_H,), jnp.int32),
    ],
)
def _hist(to_hbm, hist_out, hist_v, idx_v):
    w = _wid()
    zeros16 = jnp.zeros((L,), jnp.float32)
    ones16 = jnp.ones((L,), jnp.float32)

    def zloop(i, _):
        hist_v[pl.ds(i * L, L)] = zeros16
        return 0

    lax.fori_loop(0, NB // L, zloop, 0)
    pltpu.sync_copy(to_hbm.at[pl.ds(w * EPT_H, EPT_H)], idx_v)

    def eloop(i, _):
        plsc.addupdate_scatter(hist_v, [idx_v[pl.ds(i * L, L)]], ones16)
        return 0

    lax.fori_loop(0, EPT_H // L, eloop, 0)
    pltpu.sync_copy(hist_v, hist_out.at[w])


_TCB = 512


def _mid_body(hist_ref, ut_ref, it_ref, dis_ref, su_ref, si_ref):
    deg = jnp.sum(hist_ref[...], axis=0)
    dis_ref[0, 0, :] = jnp.where(deg > 0.0, lax.rsqrt(deg), 0.0)
    su_ref[0, 0, :] = jnp.sum(ut_ref[...], axis=1)
    si_ref[0, 0, :] = jnp.sum(it_ref[...], axis=1)


_mid = pl.pallas_call(
    _mid_body,
    grid=(NB // _TCB,),
    in_specs=[
        pl.BlockSpec((NW, _TCB), lambda g: (0, g)),
        pl.BlockSpec((_TCB, D), lambda g: (g, 0)),
        pl.BlockSpec((_TCB, D), lambda g: (g, 0)),
    ],
    out_specs=[
        pl.BlockSpec((1, 1, _TCB), lambda g: (g, 0, 0)),
        pl.BlockSpec((1, 1, _TCB), lambda g: (g, 0, 0)),
        pl.BlockSpec((1, 1, _TCB), lambda g: (g, 0, 0)),
    ],
    out_shape=[
        jax.ShapeDtypeStruct((NB // _TCB, 1, _TCB), jnp.float32),
        jax.ShapeDtypeStruct((NB // _TCB, 1, _TCB), jnp.float32),
        jax.ShapeDtypeStruct((NB // _TCB, 1, _TCB), jnp.float32),
    ],
)


@functools.partial(
    pl.kernel,
    out_type=jax.ShapeDtypeStruct((2 * NB,), jnp.float32),
    mesh=_mesh,
    compiler_params=pltpu.CompilerParams(needs_layout_passes=False),
    scratch_types=[
        pltpu.VMEM((NB,), jnp.float32),
        pltpu.VMEM((NB,), jnp.float32),
        pltpu.VMEM((NPT,), jnp.float32),
        pltpu.VMEM((NPT,), jnp.int32),
        pltpu.VMEM((NPT,), jnp.int32),
        pltpu.VMEM((NPT,), jnp.float32),
        pltpu.VMEM((NPT,), jnp.float32),
    ],
)
def _nodes(dis_hbm, su_hbm, si_hbm, f_hbm, t_hbm, g2_hbm,
           su_v, si_v, dis_v, f_v, t_v, gu_v, gi_v):
    base = _wid() * NPT
    pltpu.sync_copy(su_hbm, su_v)
    pltpu.sync_copy(si_hbm, si_v)
    pltpu.sync_copy(dis_hbm.at[pl.ds(base, NPT)], dis_v)
    pltpu.sync_copy(f_hbm.at[pl.ds(base, NPT)], f_v)
    pltpu.sync_copy(t_hbm.at[pl.ds(base, NPT)], t_v)

    def loop(i, _):
        sl = pl.ds(i * L, L)
        d = dis_v[sl]
        gu_v[sl] = d * plsc.load_gather(su_v, [f_v[sl]])
        gi_v[sl] = d * plsc.load_gather(si_v, [t_v[sl]])
        return 0

    lax.fori_loop(0, NPT // L, loop, 0)
    pltpu.sync_copy(gu_v, g2_hbm.at[pl.ds(base, NPT)])
    pltpu.sync_copy(gi_v, g2_hbm.at[pl.ds(NB + base, NPT)])


@functools.partial(
    pl.kernel,
    out_type=jax.ShapeDtypeStruct((2 * E,), jnp.float32),
    mesh=_mesh,
    compiler_params=pltpu.CompilerParams(needs_layout_passes=False),
    scratch_types=[
        pltpu.VMEM((NB,), jnp.float32),
        pltpu.VMEM((NB,), jnp.float32),
        pltpu.VMEM((CH,), jnp.int32),
        pltpu.VMEM((CH,), jnp.int32),
        pltpu.VMEM((CH,), jnp.float32),
    ],
)
def _edges(dis_hbm, g2_hbm, f_hbm, t_hbm, o2_hbm,
           dis_v, g_v, f_v, t_v, o_v):
    c = lax.axis_index("c")
    s = lax.axis_index("s")
    pltpu.sync_copy(dis_hbm, dis_v)
    pltpu.sync_copy(g2_hbm.at[pl.ds(c * NB, NB)], g_v)

    base0 = s * EPT

    def chunk(k, _):
        base = base0 + k * CH
        pltpu.sync_copy(f_hbm.at[pl.ds(base, CH)], f_v)
        pltpu.sync_copy(t_hbm.at[pl.ds(base, CH)], t_v)

        def loop(i, _):
            sl = pl.ds(i * L, L)
            o_v[sl] = (plsc.load_gather(g_v, [f_v[sl]])
                       * plsc.load_gather(dis_v, [t_v[sl]]))
            return 0

        lax.fori_loop(0, CH // L, loop, 0)
        pltpu.sync_copy(o_v, o2_hbm.at[pl.ds(c * E + base, CH)])
        return 0

    lax.fori_loop(0, EPT // CH, chunk, 0)


@jax.jit
def kernel(edge_index, user_table, item_table):
    from_ = edge_index[0]
    to_ = edge_index[1]
    to_pad = jnp.concatenate(
        [to_, jnp.full((EPAD - E,), N_NODES + 8, jnp.int32)])
    ut_pad = jnp.pad(user_table, ((0, NB - N_NODES), (0, 0)))
    it_pad = jnp.pad(item_table, ((0, NB - N_NODES), (0, 0)))

    hist = _hist(to_pad)
    dis3, su3, si3 = _mid(hist, ut_pad, it_pad)
    dis = dis3.reshape(NB)
    su = su3.reshape(NB)
    si = si3.reshape(NB)
    g2 = _nodes(dis, su, si, from_[:NB], to_[:NB])
    o2 = _edges(dis, g2, from_, to_)
    return (o2[:E], o2[E:])
